# baseline (speedup 1.0000x reference)
import functools

import jax
import jax.numpy as jnp
from jax import lax
from jax.experimental import pallas as pl
from jax.experimental.pallas import tpu as pltpu
from jax.experimental.pallas import tpu_sc as plsc

N = 2048
D = 768
E = 16
H = 768

TB = 128
NB = N // TB
P = NB + E - 1

CH = 128
NCH = N // CH


def _gate_body(x_ref, wg_ref, bg_ref, dest_ref, counts_ref, loss_ref, oh_ref):
    x = x_ref[...]
    wg = wg_ref[...]
    logits = lax.dot_general(x, wg, (((1,), (1,)), ((), ())),
                             preferred_element_type=jnp.float32)
    logits = logits + bg_ref[...]

    m = jnp.max(logits, axis=1, keepdims=True)
    ex = jnp.exp(logits - m)
    probs = ex / jnp.sum(ex, axis=1, keepdims=True)

    iota_e = lax.broadcasted_iota(jnp.int32, (N, E), 1)
    eid = jnp.min(jnp.where(logits == m, iota_e, E), axis=1, keepdims=True)
    onehot = (iota_e == eid).astype(jnp.float32)
    oh_ref[...] = onehot

    counts = jnp.sum(onehot, axis=0, keepdims=True)
    counts_ref[...] = counts

    mean_prob = jnp.sum(probs, axis=0, keepdims=True) * (1.0 / N)
    loss = jnp.sum(mean_prob * counts) * (E / N)
    loss_ref[...] = loss.reshape(1, 1)

    tri_e = (lax.broadcasted_iota(jnp.int32, (E, E), 0)
             < lax.broadcasted_iota(jnp.int32, (E, E), 1)).astype(jnp.float32)
    offv = lax.dot_general(counts, tri_e, (((1,), (0,)), ((), ())),
                           preferred_element_type=jnp.float32)

    tri_c = (lax.broadcasted_iota(jnp.int32, (CH, CH), 1)
             <= lax.broadcasted_iota(jnp.int32, (CH, CH), 0)).astype(jnp.float32)

    def body(c, carry):
        oc = oh_ref[pl.ds(c * CH, CH), :]
        cc = lax.dot_general(tri_c, oc, (((1,), (0,)), ((), ())),
                             preferred_element_type=jnp.float32)
        cc = cc + carry
        rank = jnp.sum(cc * oc, axis=1, keepdims=True) - 1.0
        offm = jnp.sum(offv * oc, axis=1, keepdims=True)
        dest_ref[pl.ds(c * CH, CH), :] = (rank + offm).astype(jnp.int32)
        return carry + jnp.sum(oc, axis=0, keepdims=True)

    lax.fori_loop(0, NCH, body, jnp.zeros((1, E), jnp.float32))


def _gate(x, wg, bg):
    return pl.pallas_call(
        _gate_body,
        out_shape=[
            jax.ShapeDtypeStruct((N, 1), jnp.int32),
            jax.ShapeDtypeStruct((1, E), jnp.float32),
            jax.ShapeDtypeStruct((1, 1), jnp.float32),
        ],
        scratch_shapes=[pltpu.VMEM((N, E), jnp.float32)],
    )(x, wg, bg.reshape(1, E))


@functools.cache
def _sc_info():
    sc = plsc.get_sparse_core_info()
    nw = sc.num_cores * sc.num_subcores
    return sc.num_cores, nw, N // nw


def _sc_scatter_rows(rows, idx):
    _NC, _NW, _BPW = _sc_info()
    mesh = plsc.VectorSubcoreMesh(core_axis_name="c", subcore_axis_name="s")

    @functools.partial(
        pl.kernel, mesh=mesh,
        out_type=jax.ShapeDtypeStruct((N, D), jnp.float32),
        scratch_types=[
            pltpu.VMEM((_BPW,), jnp.int32),
            pltpu.VMEM((_BPW, D), jnp.float32),
            pltpu.SemaphoreType.DMA,
        ],
    )
    def k(rows_hbm, idx_hbm, out_hbm, idx_v, rows_v, sem):
        wid = lax.axis_index("s") * _NC + lax.axis_index("c")
        base = wid * _BPW
        pltpu.sync_copy(idx_hbm.at[pl.ds(base, _BPW)], idx_v)
        pltpu.sync_copy(rows_hbm.at[pl.ds(base, _BPW)], rows_v)
        pltpu.async_copy(rows_v, out_hbm.at[idx_v], sem).wait()

    return k(rows, idx)


def _sc_gather_rows(table, idx):
    _NC, _NW, _BPW = _sc_info()
    mesh = plsc.VectorSubcoreMesh(core_axis_name="c", subcore_axis_name="s")

    @functools.partial(
        pl.kernel, mesh=mesh,
        out_type=jax.ShapeDtypeStruct((N, D), jnp.float32),
        scratch_types=[
            pltpu.VMEM((_BPW,), jnp.int32),
            pltpu.VMEM((_BPW, D), jnp.float32),
            pltpu.SemaphoreType.DMA,
        ],
    )
    def k(table_hbm, idx_hbm, out_hbm, idx_v, rows_v, sem):
        wid = lax.axis_index("s") * _NC + lax.axis_index("c")
        base = wid * _BPW
        pltpu.sync_copy(idx_hbm.at[pl.ds(base, _BPW)], idx_v)
        pltpu.async_copy(table_hbm.at[idx_v], rows_v, sem).wait()
        pltpu.sync_copy(rows_v, out_hbm.at[pl.ds(base, _BPW)])

    return k(table, idx)


_SQRT1_2 = 0.7071067811865476


def _ffn_body(wb_ref, we_ref, rs_ref, re_ref, ff_ref,
              x_ref, w1_ref, b1_ref, w2_ref, b2_ref, out_ref):
    w = pl.program_id(0)
    x = x_ref[...].astype(jnp.bfloat16)
    h = lax.dot_general(x, w1_ref[0], (((1,), (1,)), ((), ())),
                        preferred_element_type=jnp.float32)
    h = h + b1_ref[0]
    h = 0.5 * h * (1.0 + lax.erf(h * _SQRT1_2))
    y = lax.dot_general(h.astype(jnp.bfloat16), w2_ref[0], (((1,), (1,)), ((), ())),
                        preferred_element_type=jnp.float32)
    y = y + b2_ref[0]

    rows = lax.broadcasted_iota(jnp.int32, (TB, 1), 0)
    mask = (rows >= rs_ref[w]) & (rows < re_ref[w])
    y = jnp.where(mask, y, 0.0)

    @pl.when(ff_ref[w] == 1)
    def _init():
        out_ref[...] = y

    @pl.when(ff_ref[w] == 0)
    def _acc():
        out_ref[...] += y


def _ffn(x_sorted, w1, b1, w2, b2, wb, we, rs, re, ff):
    grid_spec = pltpu.PrefetchScalarGridSpec(
        num_scalar_prefetch=5,
        grid=(P,),
        in_specs=[
            pl.BlockSpec((TB, D), lambda w, wb, we, rs, re, ff: (wb[w], 0)),
            pl.BlockSpec((1, H, D), lambda w, wb, we, rs, re, ff: (we[w], 0, 0)),
            pl.BlockSpec((1, 1, H), lambda w, wb, we, rs, re, ff: (we[w], 0, 0)),
            pl.BlockSpec((1, D, H), lambda w, wb, we, rs, re, ff: (we[w], 0, 0)),
            pl.BlockSpec((1, 1, D), lambda w, wb, we, rs, re, ff: (we[w], 0, 0)),
        ],
        out_specs=pl.BlockSpec((TB, D), lambda w, wb, we, rs, re, ff: (wb[w], 0)),
    )
    return pl.pallas_call(
        _ffn_body,
        grid_spec=grid_spec,
        out_shape=jax.ShapeDtypeStruct((N, D), jnp.float32),
    )(wb, we, rs, re, ff, x_sorted, w1.astype(jnp.bfloat16),
      b1.reshape(E, 1, H), w2.astype(jnp.bfloat16), b2.reshape(E, 1, D))


def kernel(inputs, Wg, bg, W1, b1, W2, b2):
    b, t, d = inputs.shape
    x = inputs.reshape(N, D)

    dest2d, counts2d, loss2d = _gate(x, Wg, bg)
    dest = dest2d.reshape(N)

    counts = counts2d.reshape(E).astype(jnp.int32)
    off = jnp.concatenate([jnp.zeros((1,), jnp.int32), jnp.cumsum(counts)])
    barr = jnp.arange(NB, dtype=jnp.int32)[:, None]
    inter = (off[None, :-1] < (barr + 1) * TB) & (off[None, 1:] > barr * TB)
    flat = inter.reshape(-1)
    posl = jnp.cumsum(flat.astype(jnp.int32)) - 1
    num = jnp.sum(flat.astype(jnp.int32))
    jidx = jnp.arange(NB * E, dtype=jnp.int32)
    tgt = jnp.where(flat, posl, P)
    flatj = jnp.zeros((P + 1,), jnp.int32).at[tgt].set(jidx, mode="drop")[:P]
    valid = jnp.arange(P, dtype=jnp.int32) < num
    wb = jnp.where(valid, flatj // E, NB - 1)
    we = jnp.where(valid, flatj % E, flatj[jnp.maximum(num - 1, 0)] % E)
    rs = jnp.clip(off[we] - wb * TB, 0, TB)
    re_ = jnp.clip(off[we + 1] - wb * TB, 0, TB)
    rs = jnp.where(valid, rs, 0)
    re_ = jnp.where(valid, re_, 0)
    ff = ((jnp.arange(P, dtype=jnp.int32) == 0)
          | (wb != jnp.roll(wb, 1))).astype(jnp.int32)

    x_sorted = _sc_scatter_rows(x, dest)
    y_sorted = _ffn(x_sorted, W1, b1, W2, b2, wb, we, rs, re_, ff)
    out = _sc_gather_rows(y_sorted, dest)

    return out.reshape(b, t, d), loss2d.reshape(())


# speedup vs baseline: 1.2593x; 1.2593x over previous
import functools

import jax
import jax.numpy as jnp
from jax import lax
from jax.experimental import pallas as pl
from jax.experimental.pallas import tpu as pltpu
from jax.experimental.pallas import tpu_sc as plsc

N = 2048
D = 768
E = 16
H = 768

TB = 128
NB = N // TB
P = NB + E - 1

CH = 128
NCH = N // CH


def _gate_body(x_ref, wg_ref, bg_ref, dest_ref, counts_ref, loss_ref, oh_ref):
    x = x_ref[...]
    wg = wg_ref[...]
    logits = lax.dot_general(x, wg, (((1,), (1,)), ((), ())),
                             preferred_element_type=jnp.float32)
    logits = logits + bg_ref[...]

    m = jnp.max(logits, axis=1, keepdims=True)
    ex = jnp.exp(logits - m)
    probs = ex / jnp.sum(ex, axis=1, keepdims=True)

    iota_e = lax.broadcasted_iota(jnp.int32, (N, E), 1)
    eid = jnp.min(jnp.where(logits == m, iota_e, E), axis=1, keepdims=True)
    onehot = (iota_e == eid).astype(jnp.float32)
    oh_ref[...] = onehot

    counts = jnp.sum(onehot, axis=0, keepdims=True)
    counts_ref[...] = counts

    mean_prob = jnp.sum(probs, axis=0, keepdims=True) * (1.0 / N)
    loss = jnp.sum(mean_prob * counts) * (E / N)
    loss_ref[...] = loss.reshape(1, 1)

    tri_e = (lax.broadcasted_iota(jnp.int32, (E, E), 0)
             < lax.broadcasted_iota(jnp.int32, (E, E), 1)).astype(jnp.float32)
    offv = lax.dot_general(counts, tri_e, (((1,), (0,)), ((), ())),
                           preferred_element_type=jnp.float32)

    tri_c = (lax.broadcasted_iota(jnp.int32, (CH, CH), 1)
             <= lax.broadcasted_iota(jnp.int32, (CH, CH), 0)).astype(jnp.float32)

    def body(c, carry):
        oc = oh_ref[pl.ds(c * CH, CH), :]
        cc = lax.dot_general(tri_c, oc, (((1,), (0,)), ((), ())),
                             preferred_element_type=jnp.float32)
        cc = cc + carry
        rank = jnp.sum(cc * oc, axis=1, keepdims=True) - 1.0
        offm = jnp.sum(offv * oc, axis=1, keepdims=True)
        dest_ref[pl.ds(c * CH, CH), :] = (rank + offm).astype(jnp.int32)
        return carry + jnp.sum(oc, axis=0, keepdims=True)

    lax.fori_loop(0, NCH, body, jnp.zeros((1, E), jnp.float32))


def _gate(x, wg, bg):
    return pl.pallas_call(
        _gate_body,
        out_shape=[
            jax.ShapeDtypeStruct((N, 1), jnp.int32),
            jax.ShapeDtypeStruct((1, E), jnp.float32),
            jax.ShapeDtypeStruct((1, 1), jnp.float32),
        ],
        scratch_shapes=[pltpu.VMEM((N, E), jnp.float32)],
    )(x, wg, bg.reshape(1, E))


@functools.cache
def _sc_info():
    sc = plsc.get_sparse_core_info()
    nw = sc.num_cores * sc.num_subcores
    return sc.num_cores, nw, N // nw


def _sc_scatter_rows(rows, idx):
    _NC, _NW, _BPW = _sc_info()
    mesh = plsc.VectorSubcoreMesh(core_axis_name="c", subcore_axis_name="s")

    @functools.partial(
        pl.kernel, mesh=mesh,
        out_type=jax.ShapeDtypeStruct((N, D), jnp.float32),
        scratch_types=[
            pltpu.VMEM((_BPW,), jnp.int32),
            pltpu.VMEM((_BPW, D), jnp.float32),
            pltpu.SemaphoreType.DMA,
        ],
    )
    def k(rows_hbm, idx_hbm, out_hbm, idx_v, rows_v, sem):
        wid = lax.axis_index("s") * _NC + lax.axis_index("c")
        base = wid * _BPW
        pltpu.sync_copy(idx_hbm.at[pl.ds(base, _BPW)], idx_v)
        pltpu.sync_copy(rows_hbm.at[pl.ds(base, _BPW)], rows_v)
        pltpu.async_copy(rows_v, out_hbm.at[idx_v], sem).wait()

    return k(rows, idx)


def _sc_gather_rows(table, idx):
    _NC, _NW, _BPW = _sc_info()
    mesh = plsc.VectorSubcoreMesh(core_axis_name="c", subcore_axis_name="s")

    @functools.partial(
        pl.kernel, mesh=mesh,
        out_type=jax.ShapeDtypeStruct((N, D), jnp.float32),
        scratch_types=[
            pltpu.VMEM((_BPW,), jnp.int32),
            pltpu.VMEM((_BPW, D), jnp.float32),
            pltpu.SemaphoreType.DMA,
        ],
    )
    def k(table_hbm, idx_hbm, out_hbm, idx_v, rows_v, sem):
        wid = lax.axis_index("s") * _NC + lax.axis_index("c")
        base = wid * _BPW
        pltpu.sync_copy(idx_hbm.at[pl.ds(base, _BPW)], idx_v)
        pltpu.async_copy(table_hbm.at[idx_v], rows_v, sem).wait()
        pltpu.sync_copy(rows_v, out_hbm.at[pl.ds(base, _BPW)])

    return k(table, idx)


_SQRT1_2 = 0.7071067811865476


def _ffn_body(wb_ref, we_ref, rs_ref, re_ref, ff_ref,
              x_ref, w1_ref, b1_ref, w2_ref, b2_ref, out_ref):
    w = pl.program_id(0)
    x = x_ref[...].astype(jnp.bfloat16)
    w1 = w1_ref[0].astype(jnp.bfloat16)
    h = lax.dot_general(x, w1, (((1,), (1,)), ((), ())),
                        preferred_element_type=jnp.float32)
    h = h + b1_ref[0]
    h = 0.5 * h * (1.0 + lax.erf(h * _SQRT1_2))
    w2 = w2_ref[0].astype(jnp.bfloat16)
    y = lax.dot_general(h.astype(jnp.bfloat16), w2, (((1,), (1,)), ((), ())),
                        preferred_element_type=jnp.float32)
    y = y + b2_ref[0]

    rows = lax.broadcasted_iota(jnp.int32, (TB, 1), 0)
    mask = (rows >= rs_ref[w]) & (rows < re_ref[w])
    y = jnp.where(mask, y, 0.0)

    @pl.when(ff_ref[w] == 1)
    def _init():
        out_ref[...] = y

    @pl.when(ff_ref[w] == 0)
    def _acc():
        out_ref[...] += y


def _ffn(x_sorted, w1, b1, w2, b2, wb, we, rs, re, ff):
    grid_spec = pltpu.PrefetchScalarGridSpec(
        num_scalar_prefetch=5,
        grid=(P,),
        in_specs=[
            pl.BlockSpec((TB, D), lambda w, wb, we, rs, re, ff: (wb[w], 0)),
            pl.BlockSpec((1, H, D), lambda w, wb, we, rs, re, ff: (we[w], 0, 0)),
            pl.BlockSpec((1, 1, H), lambda w, wb, we, rs, re, ff: (we[w], 0, 0)),
            pl.BlockSpec((1, D, H), lambda w, wb, we, rs, re, ff: (we[w], 0, 0)),
            pl.BlockSpec((1, 1, D), lambda w, wb, we, rs, re, ff: (we[w], 0, 0)),
        ],
        out_specs=pl.BlockSpec((TB, D), lambda w, wb, we, rs, re, ff: (wb[w], 0)),
    )
    return pl.pallas_call(
        _ffn_body,
        grid_spec=grid_spec,
        out_shape=jax.ShapeDtypeStruct((N, D), jnp.float32),
    )(wb, we, rs, re, ff, x_sorted, w1,
      b1.reshape(E, 1, H), w2, b2.reshape(E, 1, D))


def kernel(inputs, Wg, bg, W1, b1, W2, b2):
    b, t, d = inputs.shape
    x = inputs.reshape(N, D)

    dest2d, counts2d, loss2d = _gate(x, Wg, bg)
    dest = dest2d.reshape(N)

    counts = counts2d.reshape(E).astype(jnp.int32)
    off = jnp.concatenate([jnp.zeros((1,), jnp.int32), jnp.cumsum(counts)])
    barr = jnp.arange(NB, dtype=jnp.int32)[:, None]
    inter = (off[None, :-1] < (barr + 1) * TB) & (off[None, 1:] > barr * TB)
    flat = inter.reshape(-1)
    posl = jnp.cumsum(flat.astype(jnp.int32)) - 1
    num = jnp.sum(flat.astype(jnp.int32))
    jidx = jnp.arange(NB * E, dtype=jnp.int32)
    tgt = jnp.where(flat, posl, P)
    flatj = jnp.zeros((P + 1,), jnp.int32).at[tgt].set(jidx, mode="drop")[:P]
    valid = jnp.arange(P, dtype=jnp.int32) < num
    wb = jnp.where(valid, flatj // E, NB - 1)
    we = jnp.where(valid, flatj % E, flatj[jnp.maximum(num - 1, 0)] % E)
    rs = jnp.clip(off[we] - wb * TB, 0, TB)
    re_ = jnp.clip(off[we + 1] - wb * TB, 0, TB)
    rs = jnp.where(valid, rs, 0)
    re_ = jnp.where(valid, re_, 0)
    ff = ((jnp.arange(P, dtype=jnp.int32) == 0)
          | (wb != jnp.roll(wb, 1))).astype(jnp.int32)

    x_sorted = _sc_scatter_rows(x, dest)
    y_sorted = _ffn(x_sorted, W1, b1, W2, b2, wb, we, rs, re_, ff)
    out = _sc_gather_rows(y_sorted, dest)

    return out.reshape(b, t, d), loss2d.reshape(())


# speedup vs baseline: 1.4762x; 1.1722x over previous
import functools

import jax
import jax.numpy as jnp
from jax import lax
from jax.experimental import pallas as pl
from jax.experimental.pallas import tpu as pltpu
from jax.experimental.pallas import tpu_sc as plsc

N = 2048
D = 768
E = 16
H = 768

TB = 256
NB = N // TB
P = NB + E - 1

CH = 128
NCH = N // CH


def _gate_body(x_ref, wg_ref, bg_ref, dest_ref, counts_ref, loss_ref, oh_ref):
    x = x_ref[...]
    wg = wg_ref[...]
    logits = lax.dot_general(x, wg, (((1,), (1,)), ((), ())),
                             preferred_element_type=jnp.float32)
    logits = logits + bg_ref[...]

    m = jnp.max(logits, axis=1, keepdims=True)
    ex = jnp.exp(logits - m)
    probs = ex / jnp.sum(ex, axis=1, keepdims=True)

    iota_e = lax.broadcasted_iota(jnp.int32, (N, E), 1)
    eid = jnp.min(jnp.where(logits == m, iota_e, E), axis=1, keepdims=True)
    onehot = (iota_e == eid).astype(jnp.float32)
    oh_ref[...] = onehot

    counts = jnp.sum(onehot, axis=0, keepdims=True)
    counts_ref[...] = counts

    mean_prob = jnp.sum(probs, axis=0, keepdims=True) * (1.0 / N)
    loss = jnp.sum(mean_prob * counts) * (E / N)
    loss_ref[...] = loss.reshape(1, 1)

    tri_e = (lax.broadcasted_iota(jnp.int32, (E, E), 0)
             < lax.broadcasted_iota(jnp.int32, (E, E), 1)).astype(jnp.float32)
    offv = lax.dot_general(counts, tri_e, (((1,), (0,)), ((), ())),
                           preferred_element_type=jnp.float32)

    tri_c = (lax.broadcasted_iota(jnp.int32, (CH, CH), 1)
             <= lax.broadcasted_iota(jnp.int32, (CH, CH), 0)).astype(jnp.float32)

    def body(c, carry):
        oc = oh_ref[pl.ds(c * CH, CH), :]
        cc = lax.dot_general(tri_c, oc, (((1,), (0,)), ((), ())),
                             preferred_element_type=jnp.float32)
        cc = cc + carry
        rank = jnp.sum(cc * oc, axis=1, keepdims=True) - 1.0
        offm = jnp.sum(offv * oc, axis=1, keepdims=True)
        dest_ref[pl.ds(c * CH, CH), :] = (rank + offm).astype(jnp.int32)
        return carry + jnp.sum(oc, axis=0, keepdims=True)

    lax.fori_loop(0, NCH, body, jnp.zeros((1, E), jnp.float32))


def _gate(x, wg, bg):
    return pl.pallas_call(
        _gate_body,
        out_shape=[
            jax.ShapeDtypeStruct((N, 1), jnp.int32),
            jax.ShapeDtypeStruct((1, E), jnp.float32),
            jax.ShapeDtypeStruct((1, 1), jnp.float32),
        ],
        scratch_shapes=[pltpu.VMEM((N, E), jnp.float32)],
    )(x, wg, bg.reshape(1, E))


@functools.cache
def _sc_info():
    sc = plsc.get_sparse_core_info()
    nw = sc.num_cores * sc.num_subcores
    return sc.num_cores, nw, N // nw


def _sc_scatter_rows(rows, idx):
    _NC, _NW, _BPW = _sc_info()
    mesh = plsc.VectorSubcoreMesh(core_axis_name="c", subcore_axis_name="s")

    @functools.partial(
        pl.kernel, mesh=mesh,
        out_type=jax.ShapeDtypeStruct((N, D), jnp.float32),
        scratch_types=[
            pltpu.VMEM((_BPW,), jnp.int32),
            pltpu.VMEM((_BPW, D), jnp.float32),
            pltpu.SemaphoreType.DMA,
        ],
    )
    def k(rows_hbm, idx_hbm, out_hbm, idx_v, rows_v, sem):
        wid = lax.axis_index("s") * _NC + lax.axis_index("c")
        base = wid * _BPW
        pltpu.sync_copy(idx_hbm.at[pl.ds(base, _BPW)], idx_v)
        pltpu.sync_copy(rows_hbm.at[pl.ds(base, _BPW)], rows_v)
        pltpu.async_copy(rows_v, out_hbm.at[idx_v], sem).wait()

    return k(rows, idx)


def _sc_gather_rows(table, idx):
    _NC, _NW, _BPW = _sc_info()
    mesh = plsc.VectorSubcoreMesh(core_axis_name="c", subcore_axis_name="s")

    @functools.partial(
        pl.kernel, mesh=mesh,
        out_type=jax.ShapeDtypeStruct((N, D), jnp.float32),
        scratch_types=[
            pltpu.VMEM((_BPW,), jnp.int32),
            pltpu.VMEM((_BPW, D), jnp.float32),
            pltpu.SemaphoreType.DMA,
        ],
    )
    def k(table_hbm, idx_hbm, out_hbm, idx_v, rows_v, sem):
        wid = lax.axis_index("s") * _NC + lax.axis_index("c")
        base = wid * _BPW
        pltpu.sync_copy(idx_hbm.at[pl.ds(base, _BPW)], idx_v)
        pltpu.async_copy(table_hbm.at[idx_v], rows_v, sem).wait()
        pltpu.sync_copy(rows_v, out_hbm.at[pl.ds(base, _BPW)])

    return k(table, idx)


_SQRT1_2 = 0.7071067811865476


def _ffn_body(wb_ref, we_ref, rs_ref, re_ref, ff_ref,
              x_ref, w1_ref, b1_ref, w2_ref, b2_ref, out_ref):
    w = pl.program_id(0)
    x = x_ref[...].astype(jnp.bfloat16)
    w1 = w1_ref[0].astype(jnp.bfloat16)
    h = lax.dot_general(x, w1, (((1,), (1,)), ((), ())),
                        preferred_element_type=jnp.float32)
    h = h + b1_ref[0]
    h = 0.5 * h * (1.0 + lax.erf(h * _SQRT1_2))
    w2 = w2_ref[0].astype(jnp.bfloat16)
    y = lax.dot_general(h.astype(jnp.bfloat16), w2, (((1,), (1,)), ((), ())),
                        preferred_element_type=jnp.float32)
    y = y + b2_ref[0]

    rows = lax.broadcasted_iota(jnp.int32, (TB, 1), 0)
    mask = (rows >= rs_ref[w]) & (rows < re_ref[w])
    y = jnp.where(mask, y, 0.0)

    @pl.when(ff_ref[w] == 1)
    def _init():
        out_ref[...] = y

    @pl.when(ff_ref[w] == 0)
    def _acc():
        out_ref[...] += y


def _ffn(x_sorted, w1, b1, w2, b2, wb, we, rs, re, ff):
    grid_spec = pltpu.PrefetchScalarGridSpec(
        num_scalar_prefetch=5,
        grid=(P,),
        in_specs=[
            pl.BlockSpec((TB, D), lambda w, wb, we, rs, re, ff: (wb[w], 0)),
            pl.BlockSpec((1, H, D), lambda w, wb, we, rs, re, ff: (we[w], 0, 0)),
            pl.BlockSpec((1, 1, H), lambda w, wb, we, rs, re, ff: (we[w], 0, 0)),
            pl.BlockSpec((1, D, H), lambda w, wb, we, rs, re, ff: (we[w], 0, 0)),
            pl.BlockSpec((1, 1, D), lambda w, wb, we, rs, re, ff: (we[w], 0, 0)),
        ],
        out_specs=pl.BlockSpec((TB, D), lambda w, wb, we, rs, re, ff: (wb[w], 0)),
    )
    return pl.pallas_call(
        _ffn_body,
        grid_spec=grid_spec,
        out_shape=jax.ShapeDtypeStruct((N, D), jnp.float32),
    )(wb, we, rs, re, ff, x_sorted, w1,
      b1.reshape(E, 1, H), w2, b2.reshape(E, 1, D))


def kernel(inputs, Wg, bg, W1, b1, W2, b2):
    b, t, d = inputs.shape
    x = inputs.reshape(N, D)

    dest2d, counts2d, loss2d = _gate(x, Wg, bg)
    dest = dest2d.reshape(N)

    counts = counts2d.reshape(E).astype(jnp.int32)
    off = jnp.concatenate([jnp.zeros((1,), jnp.int32), jnp.cumsum(counts)])
    barr = jnp.arange(NB, dtype=jnp.int32)[:, None]
    inter = (off[None, :-1] < (barr + 1) * TB) & (off[None, 1:] > barr * TB)
    flat = inter.reshape(-1)
    posl = jnp.cumsum(flat.astype(jnp.int32)) - 1
    num = jnp.sum(flat.astype(jnp.int32))
    jidx = jnp.arange(NB * E, dtype=jnp.int32)
    tgt = jnp.where(flat, posl, P)
    flatj = jnp.zeros((P + 1,), jnp.int32).at[tgt].set(jidx, mode="drop")[:P]
    valid = jnp.arange(P, dtype=jnp.int32) < num
    wb = jnp.where(valid, flatj // E, NB - 1)
    we = jnp.where(valid, flatj % E, flatj[jnp.maximum(num - 1, 0)] % E)
    rs = jnp.clip(off[we] - wb * TB, 0, TB)
    re_ = jnp.clip(off[we + 1] - wb * TB, 0, TB)
    rs = jnp.where(valid, rs, 0)
    re_ = jnp.where(valid, re_, 0)
    ff = ((jnp.arange(P, dtype=jnp.int32) == 0)
          | (wb != jnp.roll(wb, 1))).astype(jnp.int32)

    x_sorted = _sc_scatter_rows(x, dest)
    y_sorted = _ffn(x_sorted, W1, b1, W2, b2, wb, we, rs, re_, ff)
    out = _sc_gather_rows(y_sorted, dest)

    return out.reshape(b, t, d), loss2d.reshape(())


# speedup vs baseline: 1.6218x; 1.0986x over previous
import functools

import jax
import jax.numpy as jnp
from jax import lax
from jax.experimental import pallas as pl
from jax.experimental.pallas import tpu as pltpu
from jax.experimental.pallas import tpu_sc as plsc

N = 2048
D = 768
E = 16
H = 768

TB = 256
NB = N // TB
P = NB + E - 1

CH = 256
NCH = N // CH

NBE = NB * E
P2 = 32


def _gate_body(x_ref, wg_ref, bg_ref, dest_ref, meta_ref, loss_ref, oh_ref):
    x = x_ref[...]
    wg = wg_ref[...]
    logits = lax.dot_general(x, wg, (((1,), (1,)), ((), ())),
                             preferred_element_type=jnp.float32)
    logits = logits + bg_ref[...]

    m = jnp.max(logits, axis=1, keepdims=True)
    ex = jnp.exp(logits - m)
    probs = ex / jnp.sum(ex, axis=1, keepdims=True)

    iota_e = lax.broadcasted_iota(jnp.int32, (N, E), 1)
    eid = jnp.min(jnp.where(logits == m, iota_e, E), axis=1, keepdims=True)
    onehot = (iota_e == eid).astype(jnp.float32)
    oh_ref[...] = onehot

    counts = jnp.sum(onehot, axis=0, keepdims=True)

    mean_prob = jnp.sum(probs, axis=0, keepdims=True) * (1.0 / N)
    loss = jnp.sum(mean_prob * counts) * (E / N)
    loss_ref[...] = loss.reshape(1, 1)

    tri_e = (lax.broadcasted_iota(jnp.int32, (E, E), 0)
             < lax.broadcasted_iota(jnp.int32, (E, E), 1)).astype(jnp.float32)
    offv = lax.dot_general(counts, tri_e, (((1,), (0,)), ((), ())),
                           preferred_element_type=jnp.float32)

    tri_c = (lax.broadcasted_iota(jnp.int32, (CH, CH), 1)
             <= lax.broadcasted_iota(jnp.int32, (CH, CH), 0)).astype(jnp.float32)

    def body(c, carry):
        oc = oh_ref[pl.ds(c * CH, CH), :]
        cc = lax.dot_general(tri_c, oc, (((1,), (0,)), ((), ())),
                             preferred_element_type=jnp.float32)
        cc = cc + carry
        rank = jnp.sum(cc * oc, axis=1, keepdims=True) - 1.0
        offm = jnp.sum(offv * oc, axis=1, keepdims=True)
        dest_ref[pl.ds(c * CH, CH), :] = (rank + offm).astype(jnp.int32)
        return carry + jnp.sum(oc, axis=0, keepdims=True)

    lax.fori_loop(0, NCH, body, jnp.zeros((1, E), jnp.float32))

    off_j = jnp.concatenate([offv] * NB, axis=1).astype(jnp.int32)
    cnt_j = jnp.concatenate([counts] * NB, axis=1).astype(jnp.int32)
    end_j = off_j + cnt_j
    jlane = lax.broadcasted_iota(jnp.int32, (1, NBE), 1)
    b_j = jlane // E
    e_j = jlane - b_j * E
    bstart = b_j * TB
    flag = (off_j < bstart + TB) & (end_j > bstart)
    flagf = flag.astype(jnp.float32)

    io0 = lax.broadcasted_iota(jnp.int32, (NBE, NBE), 0)
    io1 = lax.broadcasted_iota(jnp.int32, (NBE, NBE), 1)
    tri_j = (io0 <= io1).astype(jnp.float32)
    pos = lax.dot_general(flagf, tri_j, (((1,), (0,)), ((), ())),
                          preferred_element_type=jnp.float32)
    tri_blk = ((io0 <= io1) & (io0 // E == io1 // E)).astype(jnp.float32)
    wpos = lax.dot_general(flagf, tri_blk, (((1,), (0,)), ((), ())),
                           preferred_element_type=jnp.float32)
    ff_j = flagf * (wpos == 1.0)

    rs_j = jnp.clip(off_j - bstart, 0, TB).astype(jnp.float32)
    re_j = jnp.clip(end_j - bstart, 0, TB).astype(jnp.float32)

    vals = jnp.concatenate([
        b_j.astype(jnp.float32), e_j.astype(jnp.float32),
        rs_j, re_j, ff_j,
    ], axis=0)

    idx_p = (pos - 1.0).astype(jnp.int32)
    iop = lax.broadcasted_iota(jnp.int32, (P2, NBE), 0)
    sel = ((iop == idx_p) & flag).astype(jnp.float32)
    meta = lax.dot_general(vals, sel, (((1,), (1,)), ((), ())),
                           preferred_element_type=jnp.float32)
    valid_p = jnp.sum(sel, axis=1, keepdims=True).reshape(1, P2)
    pad = (1.0 - valid_p)
    meta = jnp.concatenate([
        meta[0:1] + pad * (NB - 1),
        meta[1:2] + pad * (E - 1),
        meta[2:5],
    ], axis=0)
    meta_ref[...] = meta.astype(jnp.int32)


def _gate(x, wg, bg):
    return pl.pallas_call(
        _gate_body,
        out_shape=[
            jax.ShapeDtypeStruct((N, 1), jnp.int32),
            jax.ShapeDtypeStruct((5, P2), jnp.int32),
            jax.ShapeDtypeStruct((1, 1), jnp.float32),
        ],
        scratch_shapes=[pltpu.VMEM((N, E), jnp.float32)],
    )(x, wg, bg.reshape(1, E))


@functools.cache
def _sc_info():
    sc = plsc.get_sparse_core_info()
    nw = sc.num_cores * sc.num_subcores
    return sc.num_cores, nw, N // nw


def _sc_scatter_rows(rows, idx):
    _NC, _NW, _BPW = _sc_info()
    mesh = plsc.VectorSubcoreMesh(core_axis_name="c", subcore_axis_name="s")

    @functools.partial(
        pl.kernel, mesh=mesh,
        out_type=jax.ShapeDtypeStruct((N, D), jnp.float32),
        scratch_types=[
            pltpu.VMEM((_BPW,), jnp.int32),
            pltpu.VMEM((_BPW, D), jnp.float32),
            pltpu.SemaphoreType.DMA,
        ],
    )
    def k(rows_hbm, idx_hbm, out_hbm, idx_v, rows_v, sem):
        wid = lax.axis_index("s") * _NC + lax.axis_index("c")
        base = wid * _BPW
        pltpu.sync_copy(idx_hbm.at[pl.ds(base, _BPW)], idx_v)
        pltpu.sync_copy(rows_hbm.at[pl.ds(base, _BPW)], rows_v)
        pltpu.async_copy(rows_v, out_hbm.at[idx_v], sem).wait()

    return k(rows, idx)


def _sc_gather_rows(table, idx):
    _NC, _NW, _BPW = _sc_info()
    mesh = plsc.VectorSubcoreMesh(core_axis_name="c", subcore_axis_name="s")

    @functools.partial(
        pl.kernel, mesh=mesh,
        out_type=jax.ShapeDtypeStruct((N, D), jnp.float32),
        scratch_types=[
            pltpu.VMEM((_BPW,), jnp.int32),
            pltpu.VMEM((_BPW, D), jnp.float32),
            pltpu.SemaphoreType.DMA,
        ],
    )
    def k(table_hbm, idx_hbm, out_hbm, idx_v, rows_v, sem):
        wid = lax.axis_index("s") * _NC + lax.axis_index("c")
        base = wid * _BPW
        pltpu.sync_copy(idx_hbm.at[pl.ds(base, _BPW)], idx_v)
        pltpu.async_copy(table_hbm.at[idx_v], rows_v, sem).wait()
        pltpu.sync_copy(rows_v, out_hbm.at[pl.ds(base, _BPW)])

    return k(table, idx)


_SQRT1_2 = 0.7071067811865476


def _ffn_body(meta_ref, x_ref, w1_ref, b1_ref, w2_ref, b2_ref, out_ref):
    w = pl.program_id(0)
    x = x_ref[...].astype(jnp.bfloat16)
    w1 = w1_ref[0].astype(jnp.bfloat16)
    h = lax.dot_general(x, w1, (((1,), (1,)), ((), ())),
                        preferred_element_type=jnp.float32)
    h = h + b1_ref[0]
    h = 0.5 * h * (1.0 + lax.erf(h * _SQRT1_2))
    w2 = w2_ref[0].astype(jnp.bfloat16)
    y = lax.dot_general(h.astype(jnp.bfloat16), w2, (((1,), (1,)), ((), ())),
                        preferred_element_type=jnp.float32)
    y = y + b2_ref[0]

    rows = lax.broadcasted_iota(jnp.int32, (TB, 1), 0)
    mask = (rows >= meta_ref[2, w]) & (rows < meta_ref[3, w])
    y = jnp.where(mask, y, 0.0)

    @pl.when(meta_ref[4, w] == 1)
    def _init():
        out_ref[...] = y

    @pl.when(meta_ref[4, w] == 0)
    def _acc():
        out_ref[...] += y


def _ffn(x_sorted, w1, b1, w2, b2, meta):
    grid_spec = pltpu.PrefetchScalarGridSpec(
        num_scalar_prefetch=1,
        grid=(P,),
        in_specs=[
            pl.BlockSpec((TB, D), lambda w, meta: (meta[0, w], 0)),
            pl.BlockSpec((1, H, D), lambda w, meta: (meta[1, w], 0, 0)),
            pl.BlockSpec((1, 1, H), lambda w, meta: (meta[1, w], 0, 0)),
            pl.BlockSpec((1, D, H), lambda w, meta: (meta[1, w], 0, 0)),
            pl.BlockSpec((1, 1, D), lambda w, meta: (meta[1, w], 0, 0)),
        ],
        out_specs=pl.BlockSpec((TB, D), lambda w, meta: (meta[0, w], 0)),
    )
    return pl.pallas_call(
        _ffn_body,
        grid_spec=grid_spec,
        out_shape=jax.ShapeDtypeStruct((N, D), jnp.float32),
    )(meta, x_sorted, w1,
      b1.reshape(E, 1, H), w2, b2.reshape(E, 1, D))


def kernel(inputs, Wg, bg, W1, b1, W2, b2):
    b, t, d = inputs.shape
    x = inputs.reshape(N, D)

    dest2d, meta, loss2d = _gate(x, Wg, bg)
    dest = dest2d.reshape(N)

    x_sorted = _sc_scatter_rows(x, dest)
    y_sorted = _ffn(x_sorted, W1, b1, W2, b2, meta)
    out = _sc_gather_rows(y_sorted, dest)

    return out.reshape(b, t, d), loss2d.reshape(())
